# baseline (speedup 1.0000x reference)
import functools

import jax
import jax.numpy as jnp
from jax import lax
from jax.experimental import pallas as pl
from jax.experimental.pallas import tpu as pltpu
from jax.experimental.pallas import tpu_sc as plsc

D = 64
V = 1000000
NSAMP = 16384
TOK = 50
NCHUNK = 2
CSAMP = NSAMP // NCHUNK
CLOOK = CSAMP * TOK
NC, NS = 2, 16
NW = NC * NS
LPW = CLOOK // NW
C = 128
IBLK = 2560
NIB = LPW // IBLK
NBUF = 4
GRP = IBLK // C // NBUF


def _make_gather():
    mesh = plsc.VectorSubcoreMesh(core_axis_name="c", subcore_axis_name="s")

    @functools.partial(
        pl.kernel,
        mesh=mesh,
        out_type=jax.ShapeDtypeStruct((CLOOK, D), jnp.float32),
        scratch_types=[
            [pltpu.VMEM((IBLK,), jnp.int32) for _ in range(2)],
            [pltpu.VMEM((C, D), jnp.float32) for _ in range(NBUF)],
            [pltpu.SemaphoreType.DMA for _ in range(2)],
            [pltpu.SemaphoreType.DMA for _ in range(NBUF)],
            [pltpu.SemaphoreType.DMA for _ in range(NBUF)],
        ],
        compiler_params=pltpu.CompilerParams(
            use_tc_tiling_on_sc=False,
            disable_bounds_checks=True,
            disable_semaphore_checks=True,
        ),
    )
    def gather_kernel(x_hbm, table_hbm, out_hbm, idx_v, rows_v, isem, gsem, wsem):
        wid = lax.axis_index("s") * NC + lax.axis_index("c")
        base = wid * LPW

        icopies = [None, None]
        icopies[0] = pltpu.async_copy(
            x_hbm.at[pl.ds(base, IBLK)], idx_v[0], isem[0]
        )
        for ib in range(NIB):
            pb = ib % 2
            if ib + 1 < NIB:
                icopies[(ib + 1) % 2] = pltpu.async_copy(
                    x_hbm.at[pl.ds(base + (ib + 1) * IBLK, IBLK)],
                    idx_v[(ib + 1) % 2],
                    isem[(ib + 1) % 2],
                )
            icopies[pb].wait()
            bbase = base + ib * IBLK

            def body(g, carry, pb=pb, bbase=bbase):
                gcopies = []
                for b in range(NBUF):
                    off = (g * NBUF + b) * C
                    gcopies.append(
                        pltpu.async_copy(
                            table_hbm.at[idx_v[pb].at[pl.ds(off, C)]],
                            rows_v[b],
                            gsem[b],
                        )
                    )
                ocopies = []
                for b in range(NBUF):
                    off = (g * NBUF + b) * C
                    gcopies[b].wait()
                    ocopies.append(
                        pltpu.async_copy(
                            rows_v[b],
                            out_hbm.at[pl.ds(bbase + off, C)],
                            wsem[b],
                        )
                    )
                for cp in ocopies:
                    cp.wait()
                return carry

            lax.fori_loop(0, GRP, body, 0)

    return gather_kernel


_gather = _make_gather()


def kernel(x, embedding):
    xf = x.reshape(NCHUNK, CLOOK).astype(jnp.int32)
    halves = [
        _gather(xf[k], embedding).reshape(CSAMP, TOK, D) for k in range(NCHUNK)
    ]
    return jnp.concatenate(halves, axis=0)
